# Initial kernel scaffold; baseline (speedup 1.0000x reference)
import jax
import jax.numpy as jnp
from jax.experimental import pallas as pl


def kernel(pred_locations, pred_confidences, priors, target_boxes, target_labels):
    raise NotImplementedError("write your pallas kernel here")


# speedup vs baseline: 3.7087x; 3.7087x over previous
import jax
import jax.numpy as jnp
from jax import lax
from jax.experimental import pallas as pl

_NEG_POS_RATIO = 3
_ROWS_PER_BLOCK = 8


def _mbl_kernel(conf_ref, loc_ref, tgt_ref, lab_ref, reg_o, cls_o, np_o):
    i = pl.program_id(0)

    lab = lab_ref[...]
    R, C, N = conf_ref.shape

    c0 = conf_ref[:, 0, :]
    m = c0
    for c in range(1, C):
        m = jnp.maximum(m, conf_ref[:, c, :])
    s = jnp.zeros((R, N), jnp.float32)
    ct = jnp.zeros((R, N), jnp.float32)
    for c in range(C):
        xc = conf_ref[:, c, :]
        s = s + jnp.exp(xc - m)
        ct = ct + jnp.where(lab == c, xc, 0.0)
    lse = m + jnp.log(s)
    bg = lse - c0
    ce = lse - ct

    pos = lab > 0
    posf = pos.astype(jnp.float32)
    bits = jnp.where(pos, jnp.int32(-1), lax.bitcast_convert_type(bg, jnp.int32))
    num_pos_row = jnp.sum(pos.astype(jnp.int32), axis=1, keepdims=True)
    k = num_pos_row * _NEG_POS_RATIO

    def body1(_, carry):
        lo, hi = carry
        mid = lo + ((hi - lo) >> 1)
        cnt = jnp.sum((bits >= mid).astype(jnp.int32), axis=1, keepdims=True)
        take = cnt >= k
        return jnp.where(take, mid, lo), jnp.where(take, hi, mid)

    lo0 = jnp.zeros((R, 1), jnp.int32)
    hi0 = jnp.full((R, 1), jnp.int32(0x7F800001))
    T, _ = lax.fori_loop(0, 31, body1, (lo0, hi0))

    cnt_gt = jnp.sum((bits > T).astype(jnp.int32), axis=1, keepdims=True)
    extra = k - cnt_gt
    tie = bits == T

    iota_n = lax.broadcasted_iota(jnp.int32, (R, N), 1)

    def body2(_, carry):
        lo, hi = carry
        mid = lo + ((hi - lo) >> 1)
        cnt = jnp.sum((tie & (iota_n <= mid)).astype(jnp.int32),
                      axis=1, keepdims=True)
        take = cnt <= extra
        return jnp.where(take, mid, lo), jnp.where(take, hi, mid)

    lo0j = jnp.full((R, 1), jnp.int32(-1))
    hi0j = jnp.full((R, 1), jnp.int32(N))
    J, _ = lax.fori_loop(0, 14, body2, (lo0j, hi0j))

    sel = pos | (bits > T) | (tie & (iota_n <= J))
    cls_sum = jnp.sum(ce * sel.astype(jnp.float32))

    acc = jnp.zeros((R, N), jnp.float32)
    for c in range(loc_ref.shape[1]):
        d = loc_ref[:, c, :] - tgt_ref[:, c, :]
        ad = jnp.abs(d)
        acc = acc + jnp.where(ad < 1.0, 0.5 * d * d, ad - 0.5)
    reg_sum = jnp.sum(acc * posf)

    np_sum = jnp.sum(posf)

    @pl.when(i == 0)
    def _init():
        reg_o[...] = jnp.zeros_like(reg_o)
        cls_o[...] = jnp.zeros_like(cls_o)
        np_o[...] = jnp.zeros_like(np_o)

    reg_o[...] += reg_sum.reshape(1, 1)
    cls_o[...] += cls_sum.reshape(1, 1)
    np_o[...] += np_sum.reshape(1, 1)


@jax.jit
def kernel(pred_locations, pred_confidences, priors, target_boxes, target_labels):
    del priors
    B, N, C = pred_confidences.shape
    R = _ROWS_PER_BLOCK
    grid = (B // R,)
    labels = target_labels.astype(jnp.int32)
    confT = jnp.transpose(pred_confidences, (0, 2, 1))
    locT = jnp.transpose(pred_locations, (0, 2, 1))
    tgtT = jnp.transpose(target_boxes, (0, 2, 1))

    reg, cls, npos = pl.pallas_call(
        _mbl_kernel,
        grid=grid,
        in_specs=[
            pl.BlockSpec((R, C, N), lambda i: (i, 0, 0)),
            pl.BlockSpec((R, 4, N), lambda i: (i, 0, 0)),
            pl.BlockSpec((R, 4, N), lambda i: (i, 0, 0)),
            pl.BlockSpec((R, N), lambda i: (i, 0)),
        ],
        out_specs=[
            pl.BlockSpec((1, 1), lambda i: (0, 0)),
            pl.BlockSpec((1, 1), lambda i: (0, 0)),
            pl.BlockSpec((1, 1), lambda i: (0, 0)),
        ],
        out_shape=[
            jax.ShapeDtypeStruct((1, 1), jnp.float32),
            jax.ShapeDtypeStruct((1, 1), jnp.float32),
            jax.ShapeDtypeStruct((1, 1), jnp.float32),
        ],
    )(confT, locT, tgtT, labels)

    inv = 1.0 / npos[0, 0]
    return (reg[0, 0] * inv, cls[0, 0] * inv)
